# baseline (speedup 1.0000x reference)
import jax
import jax.numpy as jnp
from jax.experimental import pallas as pl

_B, _C, _H, _W = 16, 512, 32, 32
_D = 256


def _pos_kernel(col_ref, row_ref, out_ref):
    col_t = col_ref[0:_W, :].T
    row_t = row_ref[0:_H, :].T
    out_ref[0, 0:_D] = jnp.broadcast_to(col_t[:, None, :], (_D, _H, _W))
    out_ref[0, _D:_C] = jnp.broadcast_to(row_t[:, :, None], (_D, _H, _W))


def kernel(x, row_embed, col_embed):
    b = x.shape[0]
    return pl.pallas_call(
        _pos_kernel,
        grid=(b,),
        in_specs=[
            pl.BlockSpec(col_embed.shape, lambda i: (0, 0)),
            pl.BlockSpec(row_embed.shape, lambda i: (0, 0)),
        ],
        out_specs=pl.BlockSpec((1, _C, _H, _W), lambda i: (i, 0, 0, 0)),
        out_shape=jax.ShapeDtypeStruct((b, _C, _H, _W), jnp.float32),
    )(col_embed, row_embed)


# speedup vs baseline: 2.5773x; 2.5773x over previous
import jax
import jax.numpy as jnp
from jax import lax
from jax.experimental import pallas as pl

_B, _C, _H, _W = 16, 512, 32, 32
_D = 256
_HW = _H * _W


def _pos_kernel(col_ref, row_ref, out_ref):
    pos = lax.broadcasted_iota(jnp.int32, (_H, _HW), 1)
    sel = lax.broadcasted_iota(jnp.int32, (_H, _HW), 0)
    s_col = (lax.rem(pos, _W) == sel).astype(jnp.float32)
    s_row = (lax.div(pos, _W) == sel).astype(jnp.float32)
    dn = (((0,), (0,)), ((), ()))
    out_ref[0, 0:_D, :] = lax.dot_general(
        col_ref[0:_W, :], s_col, dn, preferred_element_type=jnp.float32)
    out_ref[0, _D:_C, :] = lax.dot_general(
        row_ref[0:_H, :], s_row, dn, preferred_element_type=jnp.float32)


def kernel(x, row_embed, col_embed):
    b = x.shape[0]
    out = pl.pallas_call(
        _pos_kernel,
        grid=(b,),
        in_specs=[
            pl.BlockSpec(col_embed.shape, lambda i: (0, 0)),
            pl.BlockSpec(row_embed.shape, lambda i: (0, 0)),
        ],
        out_specs=pl.BlockSpec((1, _C, _HW), lambda i: (i, 0, 0)),
        out_shape=jax.ShapeDtypeStruct((b, _C, _HW), jnp.float32),
    )(col_embed, row_embed)
    return out.reshape(b, _C, _H, _W)


# speedup vs baseline: 2.8019x; 1.0871x over previous
import jax
import jax.numpy as jnp
from jax import lax
from jax.experimental import pallas as pl
from jax.experimental.pallas import tpu as pltpu

_B, _C, _H, _W = 16, 512, 32, 32
_D = 256
_HW = _H * _W


def _pos_kernel(col_ref, row_ref, out_hbm, scratch, sem):
    pos = lax.broadcasted_iota(jnp.int32, (_H, _HW), 1)
    sel = lax.broadcasted_iota(jnp.int32, (_H, _HW), 0)
    s_col = (lax.rem(pos, _W) == sel).astype(jnp.float32)
    s_row = (lax.div(pos, _W) == sel).astype(jnp.float32)
    dn = (((0,), (0,)), ((), ()))
    scratch[0:_D, :] = lax.dot_general(
        col_ref[0:_W, :], s_col, dn, preferred_element_type=jnp.float32)
    scratch[_D:_C, :] = lax.dot_general(
        row_ref[0:_H, :], s_row, dn, preferred_element_type=jnp.float32)
    for b in range(_B):
        pltpu.make_async_copy(scratch, out_hbm.at[b], sem).start()
    for b in range(_B):
        pltpu.make_async_copy(scratch, out_hbm.at[b], sem).wait()


def kernel(x, row_embed, col_embed):
    b = x.shape[0]
    out = pl.pallas_call(
        _pos_kernel,
        in_specs=[
            pl.BlockSpec(memory_space=pltpu.VMEM),
            pl.BlockSpec(memory_space=pltpu.VMEM),
        ],
        out_specs=pl.BlockSpec(memory_space=pl.ANY),
        out_shape=jax.ShapeDtypeStruct((b, _C, _HW), jnp.float32),
        scratch_shapes=[
            pltpu.VMEM((_C, _HW), jnp.float32),
            pltpu.SemaphoreType.DMA,
        ],
    )(col_embed, row_embed)
    return out.reshape(b, _C, _H, _W)
